# baseline (speedup 1.0000x reference)
import functools
import math

import jax
import jax.numpy as jnp
from jax import lax
from jax.experimental import pallas as pl
from jax.experimental.pallas import tpu as pltpu
from jax.experimental.pallas import tpu_sc as plsc

D_MODEL = 64
SCALE = math.sqrt(D_MODEL)
NC = 2
NS = 16
NW = NC * NS
NGBUF = 4
NSBUF = 2
LANES = 16
BLK = 128
DBLK = D_MODEL // 8


def _make_kernel(bsz: int, seq: int):
    assert bsz == NW * BLK
    assert seq % NGBUF == 0 and seq % 8 == 0
    n_groups = seq // NGBUF

    mesh = plsc.VectorSubcoreMesh(core_axis_name="c", subcore_axis_name="s")

    @functools.partial(
        pl.kernel,
        out_type=jax.ShapeDtypeStruct((seq, DBLK, NW, 8, BLK), jnp.float32),
        mesh=mesh,
        scratch_types=[
            pltpu.VMEM((seq, BLK), jnp.int32),
            pltpu.VMEM((NGBUF, BLK, 2 * D_MODEL), jnp.float32),
            pltpu.VMEM((NSBUF, DBLK, 8, BLK), jnp.float32),
        ]
        + [pltpu.SemaphoreType.DMA] * (NGBUF + NSBUF),
        compiler_params=pltpu.CompilerParams(
            use_tc_tiling_on_sc=True, needs_layout_passes=False
        ),
    )
    def emb(xt_hbm, lut_hbm, out_hbm, idx_v, row_v, tile_v, *sems):
        gsem = sems[:NGBUF]
        osem = sems[NGBUF:]
        wid = lax.axis_index("s") * NC + lax.axis_index("c")

        pltpu.sync_copy(
            xt_hbm.at[pl.ds(0, seq), pl.ds(wid * BLK, BLK)], idx_v
        )

        iota = lax.iota(jnp.int32, LANES)
        dblk_j = [(iota + 16 * j) >> 3 for j in range(D_MODEL // LANES)]
        dsub_j = [(iota + 16 * j) & 7 for j in range(D_MODEL // LANES)]

        def start_gather(c, b):
            pltpu.async_copy(lut_hbm.at[idx_v.at[c]], row_v.at[b], gsem[b])

        def wait_gather(c, b):
            pltpu.make_async_copy(
                lut_hbm.at[idx_v.at[c]], row_v.at[b], gsem[b]
            ).wait()

        def scale_transpose(b, s):
            src = row_v.at[b]
            dst = tile_v.at[s]

            def body(r, _):
                rv = jnp.full((LANES,), r, jnp.int32)
                for j in range(D_MODEL // LANES):
                    v = src[r, pl.ds(j * LANES, LANES)] * SCALE
                    plsc.store_scatter(dst, [dblk_j[j], dsub_j[j], rv], v)
                return 0

            lax.fori_loop(0, BLK, body, 0, unroll=2)

        def start_out(c, s):
            for db in range(DBLK):
                pltpu.async_copy(
                    tile_v.at[s, db], out_hbm.at[c, db, wid], osem[s]
                )

        def wait_out(c, s):
            for db in range(DBLK):
                pltpu.make_async_copy(
                    tile_v.at[s, db], out_hbm.at[c, db, wid], osem[s]
                ).wait()

        for b in range(NGBUF):
            start_gather(b, b)

        def group(g, _):
            for b in range(NGBUF):
                c = g * NGBUF + b
                s = b % NSBUF
                wait_gather(c, b)

                if b >= NSBUF:
                    wait_out(c - NSBUF, s)
                else:

                    @pl.when(g > 0)
                    def _():
                        wait_out(c - NSBUF, s)

                scale_transpose(b, s)
                start_out(c, s)

                @pl.when(g < n_groups - 1)
                def _():
                    start_gather(c + NGBUF, b)
            return 0

        lax.fori_loop(0, n_groups, group, 0)

        for b in range(NGBUF - NSBUF, NGBUF):
            c = (n_groups - 1) * NGBUF + b
            wait_out(c, b % NSBUF)

    return emb


def kernel(x, lut):
    bsz, seq = x.shape
    vocab, d = lut.shape
    assert d == D_MODEL
    xt = x.T.astype(jnp.int32)
    lutp = jnp.pad(lut, ((0, 0), (0, d)))
    out5 = _make_kernel(bsz, seq)(xt, lutp)
    return out5.transpose(2, 4, 0, 1, 3).reshape(bsz, seq, d)


# speedup vs baseline: 1.3248x; 1.3248x over previous
import functools
import math

import jax
import jax.numpy as jnp
from jax import lax
from jax.experimental import pallas as pl
from jax.experimental.pallas import tpu as pltpu
from jax.experimental.pallas import tpu_sc as plsc

D_MODEL = 64
SCALE = math.sqrt(D_MODEL)
NC = 2
NS = 16
NW = NC * NS
NGBUF = 4
NSBUF = 2
LANES = 16
KIDX = 128


def _make_kernel(bsz: int, seq: int):
    assert bsz % NW == 0
    xrows_w = bsz // NW
    chunks_w = 2 * xrows_w
    n_groups = chunks_w // NGBUF
    assert chunks_w % NGBUF == 0 and n_groups >= 2
    kb = seq - KIDX
    assert 0 < kb <= KIDX and kb % 8 == 0
    k_of = [KIDX if b % 2 == 0 else kb for b in range(NGBUF)]

    mesh = plsc.VectorSubcoreMesh(core_axis_name="c", subcore_axis_name="s")

    @functools.partial(
        pl.kernel,
        out_type=jax.ShapeDtypeStruct((bsz, seq, 2 * D_MODEL), jnp.float32),
        mesh=mesh,
        scratch_types=[
            pltpu.VMEM((2 * xrows_w, KIDX), jnp.int32),
            pltpu.VMEM((NGBUF, KIDX, D_MODEL), jnp.float32),
            pltpu.VMEM((NSBUF, KIDX, 2 * D_MODEL), jnp.float32),
        ]
        + [pltpu.SemaphoreType.DMA] * (NGBUF + NSBUF),
        compiler_params=pltpu.CompilerParams(use_tc_tiling_on_sc=False),
    )
    def emb(x_hbm, lut_hbm, out_hbm, idx_v, row_v, sc_v, *sems):
        gsem = sems[:NGBUF]
        osem = sems[NGBUF:]
        wid = lax.axis_index("s") * NC + lax.axis_index("c")
        xrow0 = wid * xrows_w
        crow0 = 2 * xrow0

        pltpu.sync_copy(x_hbm.at[pl.ds(crow0, 2 * xrows_w)], idx_v)

        def idx_slice(c, b):
            return idx_v.at[c, pl.ds(0, k_of[b])]

        def out_slice(c, b):
            return out_hbm.at[
                xrow0 + (c >> 1), pl.ds((b % 2) * KIDX, k_of[b])
            ]

        def start_gather(c, b):
            pltpu.async_copy(
                lut_hbm.at[idx_slice(c, b)],
                row_v.at[b, pl.ds(0, k_of[b])],
                gsem[b],
            )

        def wait_gather(c, b):
            pltpu.make_async_copy(
                lut_hbm.at[idx_slice(c, b)],
                row_v.at[b, pl.ds(0, k_of[b])],
                gsem[b],
            ).wait()

        def scale(b, s):
            src = row_v.at[b]
            dst = sc_v.at[s]

            def body(r, _):
                for j in range(D_MODEL // LANES):
                    sl = pl.ds(j * LANES, LANES)
                    dst[r, sl] = src[r, sl] * SCALE
                return 0

            lax.fori_loop(0, k_of[b], body, 0, unroll=4)

        def start_out(c, b, s):
            pltpu.async_copy(
                sc_v.at[s, pl.ds(0, k_of[b])], out_slice(c, b), osem[s]
            )

        def wait_out(c, b, s):
            pltpu.make_async_copy(
                sc_v.at[s, pl.ds(0, k_of[b])], out_slice(c, b), osem[s]
            ).wait()

        for b in range(NGBUF):
            start_gather(b, b)

        def group(g, _):
            for b in range(NGBUF):
                c = g * NGBUF + b
                s = b % NSBUF
                wait_gather(c, b)

                if b >= NSBUF:
                    wait_out(c - NSBUF, b - NSBUF, s)
                else:

                    @pl.when(g > 0)
                    def _():
                        wait_out(c - NSBUF, b + NGBUF - NSBUF, s)

                scale(b, s)
                start_out(c, b, s)

                @pl.when(g < n_groups - 1)
                def _():
                    start_gather(c + NGBUF, b)
            return 0

        lax.fori_loop(0, n_groups, group, 0)

        for b in range(NGBUF - NSBUF, NGBUF):
            c = (n_groups - 1) * NGBUF + b
            wait_out(c, b, b % NSBUF)

    return emb


def kernel(x, lut):
    bsz, seq = x.shape
    vocab, d = lut.shape
    assert d == D_MODEL
    xp = jnp.pad(x.astype(jnp.int32), ((0, 0), (0, 2 * KIDX - seq)))
    xr = xp.reshape(2 * bsz, KIDX)
    out = _make_kernel(bsz, seq)(xr, lut)
    return out[..., :D_MODEL]
